# baseline (speedup 1.0000x reference)
import jax
import jax.numpy as jnp
from jax.experimental import pallas as pl
from jax.experimental.pallas import tpu as pltpu

D = 768
N_EXP = 8
TOP_K = 2
N_TOK = 32768
BT = 2048


def _router_block(h_ref, w1_ref, b1_ref, w2_ref, b2_ref, eps_ref,
                  sparse_ref, ix_ref, full_ref):
    h = h_ref[...]
    logits = jnp.dot(h, w1_ref[...],
                     preferred_element_type=jnp.float32) + b1_ref[...]
    pre = jnp.dot(h, w2_ref[...],
                  preferred_element_type=jnp.float32) + b2_ref[...]
    noisy = logits + eps_ref[...] * jax.nn.softplus(pre)

    m1 = jnp.max(noisy, axis=-1, keepdims=True)
    e = jnp.exp(noisy - m1)
    full_ref[...] = e / jnp.sum(e, axis=-1, keepdims=True)

    lanes = jax.lax.broadcasted_iota(jnp.int32, noisy.shape, 1)
    a1 = jnp.min(jnp.where(noisy == m1, lanes, N_EXP), axis=-1, keepdims=True)
    rest = jnp.where(lanes == a1, -jnp.inf, noisy)
    m2 = jnp.max(rest, axis=-1, keepdims=True)
    a2 = jnp.min(jnp.where(rest == m2, lanes, N_EXP), axis=-1, keepdims=True)

    kpos = jax.lax.broadcasted_iota(jnp.int32, (noisy.shape[0], TOP_K), 1)
    ix_ref[...] = jnp.where(kpos == 0, a1, a2)

    sel = (lanes == a1) | (lanes == a2)
    es = jnp.where(sel, e, 0.0)
    sparse_ref[...] = es / jnp.sum(es, axis=-1, keepdims=True)


def kernel(h, W_w, b_w, W_noise, b_noise):
    eps = jax.random.normal(jax.random.key(42), (N_TOK, N_EXP),
                            dtype=jnp.float32)
    w1 = W_w.T
    w2 = W_noise.T
    b1 = b_w.reshape(1, N_EXP)
    b2 = b_noise.reshape(1, N_EXP)

    grid = (N_TOK // BT,)
    sparse, ix, full = pl.pallas_call(
        _router_block,
        grid=grid,
        in_specs=[
            pl.BlockSpec((BT, D), lambda i: (i, 0)),
            pl.BlockSpec((D, N_EXP), lambda i: (0, 0)),
            pl.BlockSpec((1, N_EXP), lambda i: (0, 0)),
            pl.BlockSpec((D, N_EXP), lambda i: (0, 0)),
            pl.BlockSpec((1, N_EXP), lambda i: (0, 0)),
            pl.BlockSpec((BT, N_EXP), lambda i: (i, 0)),
        ],
        out_specs=[
            pl.BlockSpec((BT, N_EXP), lambda i: (i, 0)),
            pl.BlockSpec((BT, TOP_K), lambda i: (i, 0)),
            pl.BlockSpec((BT, N_EXP), lambda i: (i, 0)),
        ],
        out_shape=[
            jax.ShapeDtypeStruct((N_TOK, N_EXP), jnp.float32),
            jax.ShapeDtypeStruct((N_TOK, TOP_K), jnp.int32),
            jax.ShapeDtypeStruct((N_TOK, N_EXP), jnp.float32),
        ],
        compiler_params=pltpu.CompilerParams(
            dimension_semantics=("parallel",),
        ),
    )(h, w1, b1, w2, b2, eps)
    return sparse, ix, full


# speedup vs baseline: 1.0162x; 1.0162x over previous
import jax
import jax.numpy as jnp
from jax.experimental import pallas as pl
from jax.experimental.pallas import tpu as pltpu

D = 768
N_EXP = 8
TOP_K = 2
N_TOK = 32768
BT = 4096


def _router_block(h_ref, w1_ref, b1_ref, w2_ref, b2_ref, eps_ref,
                  sparse_ref, ix_ref, full_ref):
    h = h_ref[...]
    logits = jnp.dot(h, w1_ref[...],
                     preferred_element_type=jnp.float32) + b1_ref[...]
    pre = jnp.dot(h, w2_ref[...],
                  preferred_element_type=jnp.float32) + b2_ref[...]
    noisy = logits + eps_ref[...] * jax.nn.softplus(pre)

    m1 = jnp.max(noisy, axis=-1, keepdims=True)
    e = jnp.exp(noisy - m1)
    full_ref[...] = e / jnp.sum(e, axis=-1, keepdims=True)

    lanes = jax.lax.broadcasted_iota(jnp.int32, noisy.shape, 1)
    a1 = jnp.min(jnp.where(noisy == m1, lanes, N_EXP), axis=-1, keepdims=True)
    rest = jnp.where(lanes == a1, -jnp.inf, noisy)
    m2 = jnp.max(rest, axis=-1, keepdims=True)
    a2 = jnp.min(jnp.where(rest == m2, lanes, N_EXP), axis=-1, keepdims=True)

    kpos = jax.lax.broadcasted_iota(jnp.int32, (noisy.shape[0], TOP_K), 1)
    ix_ref[...] = jnp.where(kpos == 0, a1, a2)

    sel = (lanes == a1) | (lanes == a2)
    es = jnp.where(sel, e, 0.0)
    sparse_ref[...] = es / jnp.sum(es, axis=-1, keepdims=True)


def kernel(h, W_w, b_w, W_noise, b_noise):
    eps = jax.random.normal(jax.random.key(42), (N_TOK, N_EXP),
                            dtype=jnp.float32)
    w1 = W_w.T
    w2 = W_noise.T
    b1 = b_w.reshape(1, N_EXP)
    b2 = b_noise.reshape(1, N_EXP)

    grid = (N_TOK // BT,)
    sparse, ix, full = pl.pallas_call(
        _router_block,
        grid=grid,
        in_specs=[
            pl.BlockSpec((BT, D), lambda i: (i, 0)),
            pl.BlockSpec((D, N_EXP), lambda i: (0, 0)),
            pl.BlockSpec((1, N_EXP), lambda i: (0, 0)),
            pl.BlockSpec((D, N_EXP), lambda i: (0, 0)),
            pl.BlockSpec((1, N_EXP), lambda i: (0, 0)),
            pl.BlockSpec((BT, N_EXP), lambda i: (i, 0)),
        ],
        out_specs=[
            pl.BlockSpec((BT, N_EXP), lambda i: (i, 0)),
            pl.BlockSpec((BT, TOP_K), lambda i: (i, 0)),
            pl.BlockSpec((BT, N_EXP), lambda i: (i, 0)),
        ],
        out_shape=[
            jax.ShapeDtypeStruct((N_TOK, N_EXP), jnp.float32),
            jax.ShapeDtypeStruct((N_TOK, TOP_K), jnp.int32),
            jax.ShapeDtypeStruct((N_TOK, N_EXP), jnp.float32),
        ],
        compiler_params=pltpu.CompilerParams(
            dimension_semantics=("parallel",),
        ),
    )(h, w1, b1, w2, b2, eps)
    return sparse, ix, full
